# baseline (speedup 1.0000x reference)
import functools

import jax
import jax.numpy as jnp
from jax import lax
from jax.experimental import pallas as pl
from jax.experimental.pallas import tpu as pltpu
from jax.experimental.pallas import tpu_sc as plsc

_NB, _NE, _PH, _PW = 256, 128, 31, 31
_CANVAS = 200
_NPIX = _PH * _PW
_CANV_N = _CANVAS * _CANVAS
_RCH = 8
_NCHUNK = (_PH + _RCH - 1) // _RCH


def _build_kernel():
    mesh = plsc.VectorSubcoreMesh(core_axis_name="c", subcore_axis_name="s")
    info = plsc.get_sparse_core_info()
    nc, ns = info.num_cores, info.num_subcores
    nw = nc * ns
    bpw = _NB // nw

    @functools.partial(
        pl.kernel,
        mesh=mesh,
        compiler_params=pltpu.CompilerParams(needs_layout_passes=False),
        out_type=jax.ShapeDtypeStruct((_NB * _CANV_N,), jnp.float32),
        scratch_types=[
            pltpu.VMEM((_CANV_N + 64,), jnp.float32),
            pltpu.VMEM((_RCH * _PW + 8, _NE), jnp.float32),
            pltpu.VMEM((_RCH * _PW + 8, _NE), jnp.float32),
            pltpu.VMEM((_NE + 16,), jnp.int32),
            pltpu.VMEM((_NE + 16,), jnp.int32),
            pltpu.VMEM((_NE + 16,), jnp.int32),
            pltpu.SemaphoreType.DMA,
            pltpu.SemaphoreType.DMA,
        ],
    )
    def k(img_hbm, xs_hbm, ys_hbm, out_hbm,
          canvas, pbufa, pbufb, xbuf, ybuf, bbuf, sema, semb):
        wid = lax.axis_index("s") * nc + lax.axis_index("c")
        zeros16 = jnp.zeros((16,), jnp.float32)
        tail = lax.iota(jnp.int32, 16) < 15
        iota16 = lax.iota(jnp.int32, 16)
        bufs = (pbufa, pbufb)
        sems = (sema, semb)

        def chunk_copy(b, ci, buf, sem):
            p0 = ci * (_RCH * _PW)
            plen = min(_RCH * _PW, _NPIX - p0)
            return pltpu.async_copy(
                img_hbm.at[pl.ds(p0, plen), b],
                buf.at[pl.ds(0, plen)], sem)

        def per_batch(bi, _):
            b = wid * bpw + bi
            c_first = chunk_copy(b, 0, bufs[0], sems[0])

            xb = pl.multiple_of(b * _NE, 8)
            pltpu.sync_copy(xs_hbm.at[pl.ds(xb, _NE)], xbuf.at[pl.ds(0, _NE)])
            pltpu.sync_copy(ys_hbm.at[pl.ds(xb, _NE)], ybuf.at[pl.ds(0, _NE)])

            for g in range(_NE // 16):
                xv = xbuf[pl.ds(g * 16, 16)]
                yv = ybuf[pl.ds(g * 16, 16)]
                bbuf[pl.ds(g * 16, 16)] = (xv - 15) * _CANVAS + (yv - 15)

            def zbody(i, _):
                o = i * 64
                canvas[pl.ds(o, 16)] = zeros16
                canvas[pl.ds(o + 16, 16)] = zeros16
                canvas[pl.ds(o + 32, 16)] = zeros16
                canvas[pl.ds(o + 48, 16)] = zeros16
                return 0
            lax.fori_loop(0, (_CANV_N + 64) // 64, zbody, 0)

            for ci in range(_NCHUNK):
                pbuf = bufs[ci % 2]
                if ci == 0:
                    c_first.wait()
                else:
                    pending.wait()
                if ci + 1 < _NCHUNK:
                    pending = chunk_copy(b, ci + 1, bufs[(ci + 1) % 2],
                                         sems[(ci + 1) % 2])
                nrows = min(_RCH, _PH - ci * _RCH)

                def per_group(g, _, ci=ci, pbuf=pbuf, nrows=nrows):
                    g16 = g * 16
                    bvec = bbuf[pl.ds(g16, 16)] + ci * (_RCH * _CANVAS)

                    @plsc.parallel_loop(0, nrows, 1, unroll=4)
                    def per_row(r):
                        rowv = bvec + r * _CANVAS
                        pr = r * _PW
                        for c in range(_PW):
                            vv = pbuf[pr + c, pl.ds(g16, 16)]
                            plsc.addupdate_scatter(canvas, [rowv + c], vv)
                    return 0
                lax.fori_loop(0, _NE // 16, per_group, 0)

            ob = pl.multiple_of(b * _CANV_N, 8)
            pltpu.sync_copy(canvas.at[pl.ds(0, _CANV_N)],
                            out_hbm.at[pl.ds(ob, _CANV_N)])
            return 0
        lax.fori_loop(0, bpw, per_batch, 0)

    return k


_splat = _build_kernel()


def kernel(images4D, xyz):
    img = jnp.transpose(images4D, (2, 3, 0, 1)).reshape(_NPIX, _NB, _NE)
    xs = xyz[:, :, 0].reshape(_NB * _NE)
    ys = xyz[:, :, 1].reshape(_NB * _NE)
    out = _splat(img, xs, ys)
    return out.reshape(_NB, 1, _CANVAS, _CANVAS)


# speedup vs baseline: 1.0993x; 1.0993x over previous
import functools

import jax
import jax.numpy as jnp
from jax import lax
from jax.experimental import pallas as pl
from jax.experimental.pallas import tpu as pltpu
from jax.experimental.pallas import tpu_sc as plsc

_NB, _NE, _PH, _PW = 256, 128, 31, 31
_CANVAS = 200
_NPIX = _PH * _PW
_CANV_N = _CANVAS * _CANVAS
_RCH = 8
_NCHUNK = (_PH + _RCH - 1) // _RCH


def _build_kernel():
    mesh = plsc.VectorSubcoreMesh(core_axis_name="c", subcore_axis_name="s")
    info = plsc.get_sparse_core_info()
    nc, ns = info.num_cores, info.num_subcores
    nw = nc * ns
    bpw = _NB // nw

    @functools.partial(
        pl.kernel,
        mesh=mesh,
        compiler_params=pltpu.CompilerParams(needs_layout_passes=False),
        out_type=jax.ShapeDtypeStruct((_NB * _CANV_N,), jnp.float32),
        scratch_types=[
            pltpu.VMEM((_CANV_N + 64,), jnp.float32),
            pltpu.VMEM((_RCH * _PW + 8, _NE), jnp.float32),
            pltpu.VMEM((_RCH * _PW + 8, _NE), jnp.float32),
            pltpu.VMEM((_NE + 16,), jnp.int32),
            pltpu.VMEM((_NE + 16,), jnp.int32),
            pltpu.VMEM((_NE + 16,), jnp.int32),
            pltpu.SemaphoreType.DMA,
            pltpu.SemaphoreType.DMA,
        ],
    )
    def k(img_hbm, xs_hbm, ys_hbm, out_hbm,
          canvas, pbufa, pbufb, xbuf, ybuf, bbuf, sema, semb):
        wid = lax.axis_index("s") * nc + lax.axis_index("c")
        zeros16 = jnp.zeros((16,), jnp.float32)
        tail = lax.iota(jnp.int32, 16) < 15
        iota16 = lax.iota(jnp.int32, 16)
        bufs = (pbufa, pbufb)
        sems = (sema, semb)

        def chunk_copy(b, ci, buf, sem):
            p0 = ci * (_RCH * _PW)
            plen = min(_RCH * _PW, _NPIX - p0)
            return pltpu.async_copy(
                img_hbm.at[pl.ds(p0, plen), b],
                buf.at[pl.ds(0, plen)], sem)

        def per_batch(bi, _):
            b = wid * bpw + bi
            c_first = chunk_copy(b, 0, bufs[0], sems[0])

            xb = pl.multiple_of(b * _NE, 8)
            pltpu.sync_copy(xs_hbm.at[pl.ds(xb, _NE)], xbuf.at[pl.ds(0, _NE)])
            pltpu.sync_copy(ys_hbm.at[pl.ds(xb, _NE)], ybuf.at[pl.ds(0, _NE)])

            for g in range(_NE // 16):
                xv = xbuf[pl.ds(g * 16, 16)]
                yv = ybuf[pl.ds(g * 16, 16)]
                bbuf[pl.ds(g * 16, 16)] = (xv - 15) * _CANVAS + (yv - 15)

            def zbody(i, _):
                o = i * 64
                canvas[pl.ds(o, 16)] = zeros16
                canvas[pl.ds(o + 16, 16)] = zeros16
                canvas[pl.ds(o + 32, 16)] = zeros16
                canvas[pl.ds(o + 48, 16)] = zeros16
                return 0
            lax.fori_loop(0, (_CANV_N + 64) // 64, zbody, 0)

            for ci in range(_NCHUNK):
                pbuf = bufs[ci % 2]
                if ci == 0:
                    c_first.wait()
                else:
                    pending.wait()
                if ci + 1 < _NCHUNK:
                    pending = chunk_copy(b, ci + 1, bufs[(ci + 1) % 2],
                                         sems[(ci + 1) % 2])
                nrows = min(_RCH, _PH - ci * _RCH)

                def per_group(g, _, ci=ci, pbuf=pbuf, nrows=nrows):
                    g16 = g * 16
                    bvec = bbuf[pl.ds(g16, 16)] + ci * (_RCH * _CANVAS)

                    @plsc.parallel_loop(0, nrows, 1, unroll=2)
                    def per_row(r):
                        rowv = bvec + r * _CANVAS
                        pr = r * _PW
                        for c in range(_PW):
                            vv = pbuf[pr + c, pl.ds(g16, 16)]
                            plsc.addupdate_scatter(canvas, [rowv + c], vv)
                    return 0
                lax.fori_loop(0, _NE // 16, per_group, 0)

            ob = pl.multiple_of(b * _CANV_N, 8)
            pltpu.sync_copy(canvas.at[pl.ds(0, _CANV_N)],
                            out_hbm.at[pl.ds(ob, _CANV_N)])
            return 0
        lax.fori_loop(0, bpw, per_batch, 0)

    return k


_splat = _build_kernel()


def kernel(images4D, xyz):
    img = jnp.transpose(images4D, (2, 3, 0, 1)).reshape(_NPIX, _NB, _NE)
    xs = xyz[:, :, 0].reshape(_NB * _NE)
    ys = xyz[:, :, 1].reshape(_NB * _NE)
    out = _splat(img, xs, ys)
    return out.reshape(_NB, 1, _CANVAS, _CANVAS)
